# Initial kernel scaffold; baseline (speedup 1.0000x reference)
import functools

import jax
import jax.numpy as jnp
from jax import lax
from jax.experimental import pallas as pl
from jax.experimental.pallas import tpu as pltpu
from jax.experimental.pallas import tpu_sc as plsc

_D = 32
_B = 16384 * 50
_NC, _NS = 2, 16
_NW = _NC * _NS
_BPW = _B // _NW
_IR = 128
_RPG = 10
_G = _RPG * _IR
_NG = _BPW // _G
_IDX_ROWS = _BPW // _IR

_mesh = plsc.VectorSubcoreMesh(core_axis_name="c", subcore_axis_name="s")


@functools.partial(
    pl.kernel,
    mesh=_mesh,
    out_type=jax.ShapeDtypeStruct((_B, _D), jnp.float32),
    scratch_types=[
        pltpu.VMEM((_IDX_ROWS, _IR), jnp.int32),
        pltpu.VMEM((_G, _D), jnp.float32),
        pltpu.VMEM((_G, _D), jnp.float32),
        pltpu.SemaphoreType.DMA,
        pltpu.SemaphoreType.DMA,
    ],
)
def _embed_gather(idx_hbm, table_hbm, out_hbm, idx_v, rows0, rows1, sem0, sem1):
    wid = lax.axis_index("s") * _NC + lax.axis_index("c")
    base = wid * _BPW

    pltpu.sync_copy(idx_hbm.at[pl.ds(wid * _IDX_ROWS, _IDX_ROWS)], idx_v)

    rows_bufs = (rows0, rows1)
    sems = (sem0, sem1)

    def fire(g, b):
        for j in range(_RPG):
            pltpu.async_copy(
                table_hbm.at[idx_v.at[g * _RPG + j]],
                rows_bufs[b].at[pl.ds(j * _IR, _IR)],
                sems[b],
            )

    def drain(b):
        pltpu.make_async_copy(
            table_hbm.at[pl.ds(0, _G)], rows_bufs[b], sems[b]
        ).wait()

    fire(0, 0)

    def pair_body(t, carry):
        for bb in range(2):
            g = 2 * t + bb

            @pl.when(g + 1 < _NG)
            def _():
                fire(g + 1, 1 - bb)

            drain(bb)
            pltpu.sync_copy(
                rows_bufs[bb], out_hbm.at[pl.ds(base + g * _G, _G)]
            )
        return carry

    lax.fori_loop(0, _NG // 2, pair_body, 0)


def kernel(item_inputs, table):
    idx = item_inputs.reshape(_B // _IR, _IR).astype(jnp.int32)
    out = _embed_gather(idx, table)
    return out.reshape(item_inputs.shape[0], item_inputs.shape[1], _D)


# speedup vs baseline: 1.1123x; 1.1123x over previous
import functools

import jax
import jax.numpy as jnp
from jax import lax
from jax.experimental import pallas as pl
from jax.experimental.pallas import tpu as pltpu
from jax.experimental.pallas import tpu_sc as plsc

_D = 32
_B = 16384 * 50
_NC, _NS = 2, 16
_NW = _NC * _NS
_BPW = _B // _NW
_IR = 128
_RPG = 10
_G = _RPG * _IR
_NG = _BPW // _G
_IDX_ROWS = _BPW // _IR

_mesh = plsc.VectorSubcoreMesh(core_axis_name="c", subcore_axis_name="s")


@functools.partial(
    pl.kernel,
    mesh=_mesh,
    compiler_params=pltpu.CompilerParams(use_tc_tiling_on_sc=False),
    out_type=jax.ShapeDtypeStruct((_B, _D), jnp.float32),
    scratch_types=[
        pltpu.VMEM((_IDX_ROWS, _IR), jnp.int32),
        pltpu.VMEM((_G, _D), jnp.float32),
        pltpu.VMEM((_G, _D), jnp.float32),
        pltpu.SemaphoreType.DMA,
        pltpu.SemaphoreType.DMA,
    ],
)
def _embed_gather(idx_hbm, table_hbm, out_hbm, idx_v, rows0, rows1, sem0, sem1):
    wid = lax.axis_index("s") * _NC + lax.axis_index("c")
    base = wid * _BPW

    pltpu.sync_copy(idx_hbm.at[pl.ds(wid * _IDX_ROWS, _IDX_ROWS)], idx_v)

    rows_bufs = (rows0, rows1)
    sems = (sem0, sem1)

    def fire(g, b):
        for j in range(_RPG):
            pltpu.async_copy(
                table_hbm.at[idx_v.at[g * _RPG + j]],
                rows_bufs[b].at[pl.ds(j * _IR, _IR)],
                sems[b],
            )

    def drain(b):
        pltpu.make_async_copy(
            table_hbm.at[pl.ds(0, _G)], rows_bufs[b], sems[b]
        ).wait()

    fire(0, 0)

    def pair_body(t, carry):
        for bb in range(2):
            g = 2 * t + bb

            @pl.when(g + 1 < _NG)
            def _():
                fire(g + 1, 1 - bb)

            drain(bb)
            pltpu.sync_copy(
                rows_bufs[bb], out_hbm.at[pl.ds(base + g * _G, _G)]
            )
        return carry

    lax.fori_loop(0, _NG // 2, pair_body, 0)


def kernel(item_inputs, table):
    idx = item_inputs.reshape(_B // _IR, _IR).astype(jnp.int32)
    out = _embed_gather(idx, table)
    return out.reshape(item_inputs.shape[0], item_inputs.shape[1], _D)


# speedup vs baseline: 1.8064x; 1.6240x over previous
import functools

import jax
import jax.numpy as jnp
from jax import lax
from jax.experimental import pallas as pl
from jax.experimental.pallas import tpu as pltpu
from jax.experimental.pallas import tpu_sc as plsc

_D = 32
_BATCH = 16384
_HIST = 50
_NC, _NS = 2, 16
_NW = _NC * _NS
_RPW = _BATCH // _NW
_R = 16
_NG = _RPW // _R

_mesh = plsc.VectorSubcoreMesh(core_axis_name="c", subcore_axis_name="s")


@functools.partial(
    pl.kernel,
    mesh=_mesh,
    compiler_params=pltpu.CompilerParams(use_tc_tiling_on_sc=False),
    out_type=jax.ShapeDtypeStruct((_BATCH, _HIST, _D), jnp.float32),
    scratch_types=[
        pltpu.VMEM((_RPW, _HIST), jnp.int32),
        pltpu.VMEM((_R, _HIST, _D), jnp.float32),
        pltpu.VMEM((_R, _HIST, _D), jnp.float32),
        pltpu.SemaphoreType.DMA,
        pltpu.SemaphoreType.DMA,
    ],
)
def _embed_gather(idx_hbm, table_hbm, out_hbm, idx_v, rows0, rows1, sem0, sem1):
    wid = lax.axis_index("s") * _NC + lax.axis_index("c")
    row0 = wid * _RPW

    pltpu.sync_copy(idx_hbm.at[pl.ds(row0, _RPW)], idx_v)

    rows_bufs = (rows0, rows1)
    sems = (sem0, sem1)

    def fire(g, b):
        for j in range(_R):
            pltpu.async_copy(
                table_hbm.at[idx_v.at[g * _R + j]],
                rows_bufs[b].at[j],
                sems[b],
            )

    def drain(b):
        pltpu.make_async_copy(
            out_hbm.at[pl.ds(0, _R)], rows_bufs[b], sems[b]
        ).wait()

    fire(0, 0)

    def pair_body(t, carry):
        for bb in range(2):
            g = 2 * t + bb

            @pl.when(g + 1 < _NG)
            def _():
                fire(g + 1, 1 - bb)

            drain(bb)
            pltpu.sync_copy(
                rows_bufs[bb], out_hbm.at[pl.ds(row0 + g * _R, _R)]
            )
        return carry

    lax.fori_loop(0, _NG // 2, pair_body, 0)


def kernel(item_inputs, table):
    return _embed_gather(item_inputs.astype(jnp.int32), table)
